# baseline (speedup 1.0000x reference)
import functools

import jax
import jax.numpy as jnp
from jax import lax
from jax.experimental import pallas as pl
from jax.experimental.pallas import tpu as pltpu
from jax.experimental.pallas import tpu_sc as plsc

NW = 32
NC = 2
TM = 256


def _sc_gather_scatter_rows(x, tok_3d, pos_3d, P, D, C):
    nch = tok_3d.shape[1]

    @functools.partial(
        pl.kernel,
        mesh=plsc.VectorSubcoreMesh(core_axis_name="c", subcore_axis_name="s"),
        out_type=jax.ShapeDtypeStruct((P, D), x.dtype),
        scratch_types=[
            pltpu.VMEM((nch, C), jnp.int32),
            pltpu.VMEM((nch, C), jnp.int32),
            pltpu.VMEM((C, D), x.dtype),
            pltpu.VMEM((C, D), x.dtype),
            pltpu.SemaphoreType.DMA,
            pltpu.SemaphoreType.DMA,
            pltpu.SemaphoreType.DMA,
            pltpu.SemaphoreType.DMA,
        ],
    )
    def k(x_hbm, tok_hbm, pos_hbm, out_hbm, tok_v, pos_v, buf0, buf1,
          g0, g1, s0, s1):
        wid = lax.axis_index("s") * NC + lax.axis_index("c")
        pltpu.sync_copy(tok_hbm.at[wid], tok_v)
        pltpu.sync_copy(pos_hbm.at[wid], pos_v)
        bufs, gsems, ssems = (buf0, buf1), (g0, g1), (s0, s1)

        def gath(c, b):
            pltpu.async_copy(x_hbm.at[tok_v.at[c]], bufs[b], gsems[b])

        def wait_g(c, b):
            pltpu.make_async_copy(
                x_hbm.at[tok_v.at[c]], bufs[b], gsems[b]).wait()

        def stor(c, b):
            pltpu.async_copy(
                bufs[b], out_hbm.at[pos_v.at[c]], ssems[b])

        def wait_s(b):
            pltpu.make_async_copy(
                bufs[b], out_hbm.at[pos_v.at[0]], ssems[b]).wait()

        gath(0, 0)
        half = nch // 2

        def body(g, _):
            c = 2 * g

            @pl.when(g >= 1)
            def _():
                wait_s(1)

            gath(c + 1, 1)
            wait_g(c, 0)
            stor(c, 0)

            @pl.when(g < half - 1)
            def _():
                wait_s(0)
                gath(c + 2, 0)

            wait_g(c + 1, 1)
            stor(c + 1, 1)
            return 0

        lax.fori_loop(0, half, body, 0)
        wait_s(0)
        wait_s(1)

    return k(x, tok_3d, pos_3d)


def _sc_combine(cat, pos_3d, sc_3d, T, D, CT):
    nch = pos_3d.shape[1]
    L = 16

    @functools.partial(
        pl.kernel,
        mesh=plsc.VectorSubcoreMesh(core_axis_name="c", subcore_axis_name="s"),
        out_type=jax.ShapeDtypeStruct((T, D), cat.dtype),
        scratch_types=[
            pltpu.VMEM((nch, 2 * CT), jnp.int32),
            pltpu.VMEM((nch, 2 * CT, 16), cat.dtype),
            pltpu.VMEM((2 * CT, D), cat.dtype),
            pltpu.VMEM((2 * CT, D), cat.dtype),
            pltpu.VMEM((CT, D), cat.dtype),
            pltpu.VMEM((CT, D), cat.dtype),
            pltpu.SemaphoreType.DMA,
            pltpu.SemaphoreType.DMA,
            pltpu.SemaphoreType.DMA,
            pltpu.SemaphoreType.DMA,
        ],
    )
    def k(cat_hbm, pos_hbm, sc_hbm, y_hbm, idx_v, sc_v, r0, r1, o0, o1,
          g0, g1, s0, s1):
        wid = lax.axis_index("s") * NC + lax.axis_index("c")
        pltpu.sync_copy(pos_hbm.at[wid], idx_v)
        pltpu.sync_copy(sc_hbm.at[wid], sc_v)
        tw = nch * CT
        base = wid * tw
        rbufs, obufs, gsems, ssems = (r0, r1), (o0, o1), (g0, g1), (s0, s1)

        def gath(c, b):
            pltpu.async_copy(cat_hbm.at[idx_v.at[c]], rbufs[b], gsems[b])

        def wait_g(c, b):
            pltpu.make_async_copy(
                cat_hbm.at[idx_v.at[c]], rbufs[b], gsems[b]).wait()

        def stor(c, b):
            pltpu.async_copy(
                obufs[b], y_hbm.at[pl.ds(base + c * CT, CT)], ssems[b])

        def wait_s(b):
            pltpu.make_async_copy(
                obufs[b], y_hbm.at[pl.ds(base, CT)], ssems[b]).wait()

        def compute(c, b):
            rows, out_v = rbufs[b], obufs[b]
            svecs = [sc_v[c, j, :] for j in range(2 * CT)]

            def vbody(v, _):
                sl = pl.ds(v * L, L)
                for j in range(CT):
                    out_v[j, sl] = (rows[2 * j, sl] * svecs[2 * j]
                                    + rows[2 * j + 1, sl] * svecs[2 * j + 1])
                return 0

            lax.fori_loop(0, D // L, vbody, 0)

        gath(0, 0)
        half = nch // 2

        def body(g, _):
            c = 2 * g
            gath(c + 1, 1)
            wait_g(c, 0)

            @pl.when(g >= 1)
            def _():
                wait_s(0)

            compute(c, 0)
            stor(c, 0)

            @pl.when(g < half - 1)
            def _():
                gath(c + 2, 0)

            wait_g(c + 1, 1)

            @pl.when(g >= 1)
            def _():
                wait_s(1)

            compute(c + 1, 1)
            stor(c + 1, 1)
            return 0

        lax.fori_loop(0, half, body, 0)
        wait_s(0)
        wait_s(1)

    return k(cat, pos_3d, sc_3d)


def _tc_glu_body(te_ref, nu_ref, xs_ref, wg_ref, wu_ref, wd_ref, out_ref):
    t = pl.program_id(0)

    @pl.when(t < nu_ref[0])
    def _run():
        xt = xs_ref[...]
        dn = (((1,), (1,)), ((), ()))
        a = lax.dot_general(xt, wg_ref[0], dn,
                            preferred_element_type=jnp.float32)
        b = lax.dot_general(xt, wu_ref[0], dn,
                            preferred_element_type=jnp.float32)
        h = (a * lax.logistic(a)) * b
        out_ref[...] = lax.dot_general(h, wd_ref[0], dn,
                                       preferred_element_type=jnp.float32)


def kernel(x, topK_indices, topK_scores, Wg, Wu, Wd):
    T, D = x.shape
    TOPK = topK_indices.shape[1]
    E, HE, _ = Wg.shape
    N = T * TOPK
    NT = (N + E * TM) // TM
    P = NT * TM

    e = topK_indices.reshape(-1).astype(jnp.int32)
    s = topK_scores.reshape(-1)
    onehot = (e[:, None] == jnp.arange(E, dtype=jnp.int32)[None, :]).astype(
        jnp.int32)
    csum = jnp.cumsum(onehot, axis=0)
    counts = csum[-1]
    rank = jnp.take_along_axis(csum, e[:, None], axis=1)[:, 0] - 1
    padded = ((counts + TM - 1) // TM) * TM
    p_start = (jnp.cumsum(padded) - padded).astype(jnp.int32)
    pos = p_start[e] + rank
    tok = jnp.arange(N, dtype=jnp.int32) // TOPK
    tile_expert = (
        jnp.searchsorted(p_start, jnp.arange(NT, dtype=jnp.int32) * TM,
                         side="right") - 1
    ).astype(jnp.int32)
    tile_expert = jnp.clip(tile_expert, 0, E - 1)

    C = 16
    nch = N // (NW * C)
    xs = _sc_gather_scatter_rows(
        x, tok.reshape(NW, nch, C), pos.reshape(NW, nch, C), P, D, C)

    n_used = ((jnp.sum(padded) + TM - 1) // TM).astype(jnp.int32).reshape(1)
    grid_spec = pltpu.PrefetchScalarGridSpec(
        num_scalar_prefetch=2,
        grid=(NT,),
        in_specs=[
            pl.BlockSpec((TM, D), lambda t, te, nu: (t, 0)),
            pl.BlockSpec((1, HE, D), lambda t, te, nu: (te[t], 0, 0)),
            pl.BlockSpec((1, HE, D), lambda t, te, nu: (te[t], 0, 0)),
            pl.BlockSpec((1, D, HE), lambda t, te, nu: (te[t], 0, 0)),
        ],
        out_specs=pl.BlockSpec((TM, D), lambda t, te, nu: (t, 0)),
    )
    cat = pl.pallas_call(
        _tc_glu_body,
        grid_spec=grid_spec,
        out_shape=jax.ShapeDtypeStruct((P, D), jnp.float32),
    )(tile_expert, n_used, xs, Wg, Wu, Wd)

    CT = 4
    nct = T // (NW * CT)
    pos_3d = pos.reshape(NW, nct, CT * TOPK)
    sc_4d = jnp.broadcast_to(
        s.reshape(NW, nct, CT * TOPK, 1), (NW, nct, CT * TOPK, 16))
    y = _sc_combine(cat, pos_3d, sc_4d, T, D, CT)
    return y


# speedup vs baseline: 1.0313x; 1.0313x over previous
import functools

import jax
import jax.numpy as jnp
from jax import lax
from jax.experimental import pallas as pl
from jax.experimental.pallas import tpu as pltpu
from jax.experimental.pallas import tpu_sc as plsc

NW = 32
NC = 2
TM = 256


def _sc_gather_scatter_rows(x, tok_3d, pos_3d, P, D, C):
    nch = tok_3d.shape[1]

    @functools.partial(
        pl.kernel,
        mesh=plsc.VectorSubcoreMesh(core_axis_name="c", subcore_axis_name="s"),
        out_type=jax.ShapeDtypeStruct((P, D), x.dtype),
        scratch_types=[
            pltpu.VMEM((nch, C), jnp.int32),
            pltpu.VMEM((nch, C), jnp.int32),
            pltpu.VMEM((C, D), x.dtype),
            pltpu.VMEM((C, D), x.dtype),
            pltpu.SemaphoreType.DMA,
            pltpu.SemaphoreType.DMA,
            pltpu.SemaphoreType.DMA,
            pltpu.SemaphoreType.DMA,
        ],
    )
    def k(x_hbm, tok_hbm, pos_hbm, out_hbm, tok_v, pos_v, buf0, buf1,
          g0, g1, s0, s1):
        wid = lax.axis_index("s") * NC + lax.axis_index("c")
        pltpu.sync_copy(tok_hbm.at[wid], tok_v)
        pltpu.sync_copy(pos_hbm.at[wid], pos_v)
        bufs, gsems, ssems = (buf0, buf1), (g0, g1), (s0, s1)

        def gath(c, b):
            pltpu.async_copy(x_hbm.at[tok_v.at[c]], bufs[b], gsems[b])

        def wait_g(c, b):
            pltpu.make_async_copy(
                x_hbm.at[tok_v.at[c]], bufs[b], gsems[b]).wait()

        def stor(c, b):
            pltpu.async_copy(
                bufs[b], out_hbm.at[pos_v.at[c]], ssems[b])

        def wait_s(b):
            pltpu.make_async_copy(
                bufs[b], out_hbm.at[pos_v.at[0]], ssems[b]).wait()

        gath(0, 0)
        half = nch // 2

        def body(g, _):
            c = 2 * g

            @pl.when(g >= 1)
            def _():
                wait_s(1)

            gath(c + 1, 1)
            wait_g(c, 0)
            stor(c, 0)

            @pl.when(g < half - 1)
            def _():
                wait_s(0)
                gath(c + 2, 0)

            wait_g(c + 1, 1)
            stor(c + 1, 1)
            return 0

        lax.fori_loop(0, half, body, 0)
        wait_s(0)
        wait_s(1)

    return k(x, tok_3d, pos_3d)


def _sc_combine(cat, pos_3d, sc_3d, T, D, CT):
    nch = pos_3d.shape[1]
    L = 16

    @functools.partial(
        pl.kernel,
        mesh=plsc.VectorSubcoreMesh(core_axis_name="c", subcore_axis_name="s"),
        out_type=jax.ShapeDtypeStruct((T, D), cat.dtype),
        scratch_types=[
            pltpu.VMEM((nch, 2 * CT), jnp.int32),
            pltpu.VMEM((nch, 2 * CT, 16), cat.dtype),
            pltpu.VMEM((2 * CT, D), cat.dtype),
            pltpu.VMEM((2 * CT, D), cat.dtype),
            pltpu.VMEM((CT, D), cat.dtype),
            pltpu.VMEM((CT, D), cat.dtype),
            pltpu.SemaphoreType.DMA,
            pltpu.SemaphoreType.DMA,
            pltpu.SemaphoreType.DMA,
            pltpu.SemaphoreType.DMA,
        ],
    )
    def k(cat_hbm, pos_hbm, sc_hbm, y_hbm, idx_v, sc_v, r0, r1, o0, o1,
          g0, g1, s0, s1):
        wid = lax.axis_index("s") * NC + lax.axis_index("c")
        pltpu.sync_copy(pos_hbm.at[wid], idx_v)
        pltpu.sync_copy(sc_hbm.at[wid], sc_v)
        tw = nch * CT
        base = wid * tw
        rbufs, obufs, gsems, ssems = (r0, r1), (o0, o1), (g0, g1), (s0, s1)

        def gath(c, b):
            pltpu.async_copy(cat_hbm.at[idx_v.at[c]], rbufs[b], gsems[b])

        def wait_g(c, b):
            pltpu.make_async_copy(
                cat_hbm.at[idx_v.at[c]], rbufs[b], gsems[b]).wait()

        def stor(c, b):
            pltpu.async_copy(
                obufs[b], y_hbm.at[pl.ds(base + c * CT, CT)], ssems[b])

        def wait_s(b):
            pltpu.make_async_copy(
                obufs[b], y_hbm.at[pl.ds(base, CT)], ssems[b]).wait()

        def compute(c, b):
            rows, out_v = rbufs[b], obufs[b]
            svecs = [sc_v[c, j, :] for j in range(2 * CT)]

            def vbody(v, _):
                sl = pl.ds(v * L, L)
                for j in range(CT):
                    out_v[j, sl] = (rows[2 * j, sl] * svecs[2 * j]
                                    + rows[2 * j + 1, sl] * svecs[2 * j + 1])
                return 0

            lax.fori_loop(0, D // L, vbody, 0)

        gath(0, 0)
        half = nch // 2

        def body(g, _):
            c = 2 * g
            gath(c + 1, 1)
            wait_g(c, 0)

            @pl.when(g >= 1)
            def _():
                wait_s(0)

            compute(c, 0)
            stor(c, 0)

            @pl.when(g < half - 1)
            def _():
                gath(c + 2, 0)

            wait_g(c + 1, 1)

            @pl.when(g >= 1)
            def _():
                wait_s(1)

            compute(c + 1, 1)
            stor(c + 1, 1)
            return 0

        lax.fori_loop(0, half, body, 0)
        wait_s(0)
        wait_s(1)

    return k(cat, pos_3d, sc_3d)


def _tc_glu_body(te_ref, nu_ref, xs_ref, wg_ref, wu_ref, wd_ref, out_ref):
    t = pl.program_id(0)

    @pl.when(t < nu_ref[0])
    def _run():
        xt = xs_ref[...]
        dn = (((1,), (1,)), ((), ()))
        a = lax.dot_general(xt, wg_ref[0], dn,
                            preferred_element_type=jnp.float32)
        b = lax.dot_general(xt, wu_ref[0], dn,
                            preferred_element_type=jnp.float32)
        h = (a * lax.logistic(a)) * b
        out_ref[...] = lax.dot_general(h, wd_ref[0], dn,
                                       preferred_element_type=jnp.float32)


def kernel(x, topK_indices, topK_scores, Wg, Wu, Wd):
    T, D = x.shape
    TOPK = topK_indices.shape[1]
    E, HE, _ = Wg.shape
    N = T * TOPK
    NT = (N + E * TM) // TM
    P = NT * TM

    e = topK_indices.reshape(-1).astype(jnp.int32)
    s = topK_scores.reshape(-1)
    onehot = (e[:, None] == jnp.arange(E, dtype=jnp.int32)[None, :]).astype(
        jnp.int32)
    csum = jnp.cumsum(onehot, axis=0)
    counts = csum[-1]
    padded = ((counts + TM - 1) // TM) * TM
    p_start = (jnp.cumsum(padded) - padded).astype(jnp.int32)
    pos = jnp.sum(onehot * (csum + p_start[None, :]), axis=1) - 1
    tok = jnp.arange(N, dtype=jnp.int32) // TOPK
    bases = jnp.arange(NT, dtype=jnp.int32) * TM
    tile_expert = jnp.clip(
        jnp.sum((p_start[None, :] <= bases[:, None]).astype(jnp.int32),
                axis=1) - 1, 0, E - 1).astype(jnp.int32)

    C = 16
    nch = N // (NW * C)
    xs = _sc_gather_scatter_rows(
        x, tok.reshape(NW, nch, C), pos.reshape(NW, nch, C), P, D, C)

    n_used = ((jnp.sum(padded) + TM - 1) // TM).astype(jnp.int32).reshape(1)
    grid_spec = pltpu.PrefetchScalarGridSpec(
        num_scalar_prefetch=2,
        grid=(NT,),
        in_specs=[
            pl.BlockSpec((TM, D), lambda t, te, nu: (t, 0)),
            pl.BlockSpec((1, HE, D), lambda t, te, nu: (te[t], 0, 0)),
            pl.BlockSpec((1, HE, D), lambda t, te, nu: (te[t], 0, 0)),
            pl.BlockSpec((1, D, HE), lambda t, te, nu: (te[t], 0, 0)),
        ],
        out_specs=pl.BlockSpec((TM, D), lambda t, te, nu: (t, 0)),
    )
    cat = pl.pallas_call(
        _tc_glu_body,
        grid_spec=grid_spec,
        out_shape=jax.ShapeDtypeStruct((P, D), jnp.float32),
    )(tile_expert, n_used, xs, Wg, Wu, Wd)

    CT = 4
    nct = T // (NW * CT)
    pos_3d = pos.reshape(NW, nct, CT * TOPK)
    sc_4d = jnp.broadcast_to(
        s.reshape(NW, nct, CT * TOPK, 1), (NW, nct, CT * TOPK, 16))
    y = _sc_combine(cat, pos_3d, sc_4d, T, D, CT)
    return y


# speedup vs baseline: 1.1691x; 1.1337x over previous
import functools

import jax
import jax.numpy as jnp
from jax import lax
from jax.experimental import pallas as pl
from jax.experimental.pallas import tpu as pltpu
from jax.experimental.pallas import tpu_sc as plsc

NW = 32
NC = 2
TM = 256


def _sc_gather_rows(x, idx_3d, P, D, C):
    nch = idx_3d.shape[1]

    @functools.partial(
        pl.kernel,
        mesh=plsc.VectorSubcoreMesh(core_axis_name="c", subcore_axis_name="s"),
        out_type=jax.ShapeDtypeStruct((P, D), x.dtype),
        scratch_types=[
            pltpu.VMEM((nch, C), jnp.int32),
            pltpu.VMEM((C, D), x.dtype),
            pltpu.VMEM((C, D), x.dtype),
            pltpu.SemaphoreType.DMA,
            pltpu.SemaphoreType.DMA,
            pltpu.SemaphoreType.DMA,
            pltpu.SemaphoreType.DMA,
        ],
    )
    def k(x_hbm, idx_hbm, out_hbm, idx_v, buf0, buf1, g0, g1, s0, s1):
        wid = lax.axis_index("s") * NC + lax.axis_index("c")
        pltpu.sync_copy(idx_hbm.at[wid], idx_v)
        per_w = nch * C
        base = wid * per_w
        bufs, gsems, ssems = (buf0, buf1), (g0, g1), (s0, s1)

        def gath(c, b):
            pltpu.async_copy(x_hbm.at[idx_v.at[c]], bufs[b], gsems[b])

        def wait_g(c, b):
            pltpu.make_async_copy(
                x_hbm.at[idx_v.at[c]], bufs[b], gsems[b]).wait()

        def stor(c, b):
            pltpu.async_copy(
                bufs[b], out_hbm.at[pl.ds(base + c * C, C)], ssems[b])

        def wait_s(b):
            pltpu.make_async_copy(
                bufs[b], out_hbm.at[pl.ds(base, C)], ssems[b]).wait()

        gath(0, 0)
        half = nch // 2

        def body(g, _):
            c = 2 * g

            @pl.when(g >= 1)
            def _():
                wait_s(1)

            gath(c + 1, 1)
            wait_g(c, 0)
            stor(c, 0)

            @pl.when(g < half - 1)
            def _():
                wait_s(0)
                gath(c + 2, 0)

            wait_g(c + 1, 1)
            stor(c + 1, 1)
            return 0

        lax.fori_loop(0, half, body, 0)
        wait_s(0)
        wait_s(1)

    return k(x, idx_3d)


def _sc_combine(cat, pos_3d, T, D, CT):
    nch = pos_3d.shape[1]
    L = 16

    @functools.partial(
        pl.kernel,
        mesh=plsc.VectorSubcoreMesh(core_axis_name="c", subcore_axis_name="s"),
        out_type=jax.ShapeDtypeStruct((T, D), cat.dtype),
        scratch_types=[
            pltpu.VMEM((nch, 2 * CT), jnp.int32),
            pltpu.VMEM((2 * CT, D), cat.dtype),
            pltpu.VMEM((2 * CT, D), cat.dtype),
            pltpu.VMEM((CT, D), cat.dtype),
            pltpu.VMEM((CT, D), cat.dtype),
            pltpu.SemaphoreType.DMA,
            pltpu.SemaphoreType.DMA,
            pltpu.SemaphoreType.DMA,
            pltpu.SemaphoreType.DMA,
        ],
    )
    def k(cat_hbm, pos_hbm, y_hbm, idx_v, r0, r1, o0, o1, g0, g1, s0, s1):
        wid = lax.axis_index("s") * NC + lax.axis_index("c")
        pltpu.sync_copy(pos_hbm.at[wid], idx_v)
        tw = nch * CT
        base = wid * tw
        rbufs, obufs, gsems, ssems = (r0, r1), (o0, o1), (g0, g1), (s0, s1)

        def gath(c, b):
            pltpu.async_copy(cat_hbm.at[idx_v.at[c]], rbufs[b], gsems[b])

        def wait_g(c, b):
            pltpu.make_async_copy(
                cat_hbm.at[idx_v.at[c]], rbufs[b], gsems[b]).wait()

        def stor(c, b):
            pltpu.async_copy(
                obufs[b], y_hbm.at[pl.ds(base + c * CT, CT)], ssems[b])

        def wait_s(b):
            pltpu.make_async_copy(
                obufs[b], y_hbm.at[pl.ds(base, CT)], ssems[b]).wait()

        def compute(b):
            rows, out_v = rbufs[b], obufs[b]

            def vbody(v, _):
                sl = pl.ds(v * L, L)
                for j in range(CT):
                    out_v[j, sl] = rows[2 * j, sl] + rows[2 * j + 1, sl]
                return 0

            lax.fori_loop(0, D // L, vbody, 0)

        gath(0, 0)
        half = nch // 2

        def body(g, _):
            c = 2 * g
            gath(c + 1, 1)
            wait_g(c, 0)

            @pl.when(g >= 1)
            def _():
                wait_s(0)

            compute(0)
            stor(c, 0)

            @pl.when(g < half - 1)
            def _():
                gath(c + 2, 0)

            wait_g(c + 1, 1)

            @pl.when(g >= 1)
            def _():
                wait_s(1)

            compute(1)
            stor(c + 1, 1)
            return 0

        lax.fori_loop(0, half, body, 0)
        wait_s(0)
        wait_s(1)

    return k(cat, pos_3d)


def _tc_glu_body(te_ref, nu_ref, xs_ref, wg_ref, wu_ref, wd_ref, s_ref,
                 out_ref):
    t = pl.program_id(0)

    @pl.when(t < nu_ref[0])
    def _run():
        xt = xs_ref[...]
        dn = (((1,), (1,)), ((), ()))
        a = lax.dot_general(xt, wg_ref[0], dn,
                            preferred_element_type=jnp.float32)
        b = lax.dot_general(xt, wu_ref[0], dn,
                            preferred_element_type=jnp.float32)
        h = (a * lax.logistic(a)) * b
        o = lax.dot_general(h, wd_ref[0], dn,
                            preferred_element_type=jnp.float32)
        sv = s_ref[0, 0, :]
        out_ref[...] = o * sv[:, None]


def kernel(x, topK_indices, topK_scores, Wg, Wu, Wd):
    T, D = x.shape
    TOPK = topK_indices.shape[1]
    E, HE, _ = Wg.shape
    N = T * TOPK
    NT = (N + E * TM) // TM
    P = NT * TM

    e = topK_indices.reshape(-1).astype(jnp.int32)
    s = topK_scores.reshape(-1)
    onehot = (e[:, None] == jnp.arange(E, dtype=jnp.int32)[None, :]).astype(
        jnp.int32)
    csum = jnp.cumsum(onehot, axis=0)
    counts = csum[-1]
    padded = ((counts + TM - 1) // TM) * TM
    p_start = (jnp.cumsum(padded) - padded).astype(jnp.int32)
    pos = jnp.sum(onehot * (csum + p_start[None, :]), axis=1) - 1
    tok = jnp.arange(N, dtype=jnp.int32) // TOPK
    src_token = (jnp.arange(P, dtype=jnp.int32) % T).at[pos].set(tok)
    pad_scores = jnp.zeros((P,), x.dtype).at[pos].set(s)
    bases = jnp.arange(NT, dtype=jnp.int32) * TM
    tile_expert = jnp.clip(
        jnp.sum((p_start[None, :] <= bases[:, None]).astype(jnp.int32),
                axis=1) - 1, 0, E - 1).astype(jnp.int32)

    C = 16
    xs = _sc_gather_rows(x, src_token.reshape(NW, P // (NW * C), C), P, D, C)

    n_used = ((jnp.sum(padded) + TM - 1) // TM).astype(jnp.int32).reshape(1)
    grid_spec = pltpu.PrefetchScalarGridSpec(
        num_scalar_prefetch=2,
        grid=(NT,),
        in_specs=[
            pl.BlockSpec((TM, D), lambda t, te, nu: (t, 0)),
            pl.BlockSpec((1, HE, D), lambda t, te, nu: (te[t], 0, 0)),
            pl.BlockSpec((1, HE, D), lambda t, te, nu: (te[t], 0, 0)),
            pl.BlockSpec((1, D, HE), lambda t, te, nu: (te[t], 0, 0)),
            pl.BlockSpec((1, 1, TM), lambda t, te, nu: (t, 0, 0)),
        ],
        out_specs=pl.BlockSpec((TM, D), lambda t, te, nu: (t, 0)),
    )
    cat = pl.pallas_call(
        _tc_glu_body,
        grid_spec=grid_spec,
        out_shape=jax.ShapeDtypeStruct((P, D), jnp.float32),
    )(tile_expert, n_used, xs, Wg, Wu, Wd, pad_scores.reshape(NT, 1, TM))

    CT = 8
    pos_by_tok = pos.reshape(T, TOPK)
    pos_3d = pos_by_tok.reshape(NW, T // (NW * CT), CT * TOPK)
    y = _sc_combine(cat, pos_3d, T, D, CT)
    return y
